# baseline (speedup 1.0000x reference)
import jax
import jax.numpy as jnp
from jax.experimental import pallas as pl

N_ENT = 50000
CH = 64
N_USERS = 1024
N_REL = 24
RS, RE = 42033, 44630


_MM_KB = 2048


def _mm_body(im_ref, emb_ref, out_ref):
    k = pl.program_id(0)

    @pl.when(k == 0)
    def _():
        out_ref[...] = jnp.zeros_like(out_ref)

    col = k * _MM_KB + jax.lax.broadcasted_iota(jnp.int32, (_MM_KB, 1), 0)
    valid = col < N_ENT
    im = jnp.where(valid.T, im_ref[...], 0.0)
    emb = jnp.where(valid, emb_ref[...], 0.0)
    out_ref[...] += jnp.dot(im, emb, preferred_element_type=jnp.float32)


def _user_matmul(interact_mat, emb):
    grid = (pl.cdiv(N_ENT, _MM_KB),)
    return pl.pallas_call(
        _mm_body,
        grid=grid,
        in_specs=[
            pl.BlockSpec((N_USERS, _MM_KB), lambda k: (0, k)),
            pl.BlockSpec((_MM_KB, CH), lambda k: (k, 0)),
        ],
        out_specs=pl.BlockSpec((N_USERS, CH), lambda k: (0, 0)),
        out_shape=jax.ShapeDtypeStruct((N_USERS, CH), jnp.float32),
    )(interact_mat, emb)


def kernel(entity_emb, user_emb, edge_index, edge_type, interact_mat,
           region_weight_matrix, weight):
    det = jax.lax.stop_gradient(entity_emb)
    new_entity_vectors = det.at[RS:RE].set(
        jnp.matmul(region_weight_matrix, det[RS:RE]))
    entity_emb = entity_emb * 0.8 + new_entity_vectors * 0.2
    head = edge_index[0]
    tail = edge_index[1]
    rel_idx = (edge_type - 1) % weight.shape[0]
    edge_relation_emb = weight[rel_idx]
    neigh_relation_emb = entity_emb[tail] * edge_relation_emb
    s = jax.ops.segment_sum(neigh_relation_emb, head, num_segments=N_ENT)
    cnt = jax.ops.segment_sum(jnp.ones((head.shape[0],), jnp.float32), head,
                              num_segments=N_ENT)
    entity_agg = s / jnp.clip(cnt, 1.0, None)[:, None]
    user_agg = _user_matmul(interact_mat, entity_emb)
    return (entity_agg, user_agg)


# speedup vs baseline: 1.0269x; 1.0269x over previous
import jax
import jax.numpy as jnp
from jax import lax
from jax.experimental import pallas as pl

N_ENT = 50000
CH = 64
N_USERS = 1024
N_REL = 24
RS, RE = 42033, 44630
RN = RE - RS


_RKB = 512


def _region_body(rw_ref, det_ref, emb_ref, out_ref):
    k = pl.program_id(0)

    @pl.when(k == 0)
    def _():
        out_ref[...] = 0.8 * emb_ref[...]

    col = k * _RKB + lax.broadcasted_iota(jnp.int32, (_RKB, 1), 0)
    valid = col < RN
    rw = jnp.where(valid.T, rw_ref[...], 0.0)
    det = jnp.where(valid, det_ref[...], 0.0)
    out_ref[...] += 0.2 * jnp.dot(rw, det, preferred_element_type=jnp.float32)


def _region_blend(region_weight_matrix, emb_region):
    return pl.pallas_call(
        _region_body,
        grid=(pl.cdiv(RN, _RKB),),
        in_specs=[
            pl.BlockSpec((RN, _RKB), lambda k: (0, k)),
            pl.BlockSpec((_RKB, CH), lambda k: (k, 0)),
            pl.BlockSpec((RN, CH), lambda k: (0, 0)),
        ],
        out_specs=pl.BlockSpec((RN, CH), lambda k: (0, 0)),
        out_shape=jax.ShapeDtypeStruct((RN, CH), jnp.float32),
    )(region_weight_matrix, emb_region, emb_region)


_MM_KB = 2048


def _mm_body(im_ref, emb_ref, out_ref):
    k = pl.program_id(0)

    @pl.when(k == 0)
    def _():
        out_ref[...] = jnp.zeros_like(out_ref)

    col = k * _MM_KB + lax.broadcasted_iota(jnp.int32, (_MM_KB, 1), 0)
    valid = col < N_ENT
    im = jnp.where(valid.T, im_ref[...], 0.0)
    emb = jnp.where(valid, emb_ref[...], 0.0)
    out_ref[...] += jnp.dot(im, emb, preferred_element_type=jnp.float32)


def _user_matmul(interact_mat, emb):
    return pl.pallas_call(
        _mm_body,
        grid=(pl.cdiv(N_ENT, _MM_KB),),
        in_specs=[
            pl.BlockSpec((N_USERS, _MM_KB), lambda k: (0, k)),
            pl.BlockSpec((_MM_KB, CH), lambda k: (k, 0)),
        ],
        out_specs=pl.BlockSpec((N_USERS, CH), lambda k: (0, 0)),
        out_shape=jax.ShapeDtypeStruct((N_USERS, CH), jnp.float32),
    )(interact_mat, emb)


def kernel(entity_emb, user_emb, edge_index, edge_type, interact_mat,
           region_weight_matrix, weight):
    region_out = _region_blend(region_weight_matrix, entity_emb[RS:RE])
    emb2 = jnp.concatenate([entity_emb[:RS], region_out, entity_emb[RE:]],
                           axis=0)

    head = edge_index[0]
    tail = edge_index[1]
    rel_idx = (edge_type - 1) % weight.shape[0]
    edge_relation_emb = weight[rel_idx]
    neigh_relation_emb = emb2[tail] * edge_relation_emb
    s = jax.ops.segment_sum(neigh_relation_emb, head, num_segments=N_ENT)
    cnt = jax.ops.segment_sum(jnp.ones((head.shape[0],), jnp.float32), head,
                              num_segments=N_ENT)
    entity_agg = s / jnp.clip(cnt, 1.0, None)[:, None]

    user_agg = _user_matmul(interact_mat, emb2)
    return (entity_agg, user_agg)
